# initial kernel scaffold (unmeasured)
import jax
import jax.numpy as jnp
from jax import lax
from jax.experimental import pallas as pl
from jax.experimental.pallas import tpu as pltpu

_sem_signal = getattr(pltpu, "semaphore_signal", None) or pl.semaphore_signal
_sem_wait = getattr(pltpu, "semaphore_wait", None) or pl.semaphore_wait
_DevIdType = getattr(pltpu, "DeviceIdType", None) or pl.DeviceIdType


def kernel(Q, K, V):
    b, s, h, d = Q.shape
    scale = d ** -0.5

    qt = jnp.transpose(Q[0], (1, 0, 2))
    kt = jnp.transpose(K[0], (1, 0, 2))
    vt = jnp.transpose(V[0], (1, 0, 2))

    def body(q_ref, k_ref, v_ref, o_ref, kr_ref, vr_ref, sems):
        my_x = lax.axis_index("x")
        my_y = lax.axis_index("y")
        my_z = lax.axis_index("z")
        partner = (1 - my_x, my_y, my_z)

        barrier = pltpu.get_barrier_semaphore()
        _sem_signal(barrier, inc=1, device_id=partner,
                    device_id_type=_DevIdType.MESH)
        _sem_wait(barrier, 1)

        rk = pltpu.make_async_remote_copy(
            src_ref=k_ref, dst_ref=kr_ref,
            send_sem=sems.at[0], recv_sem=sems.at[1],
            device_id=partner, device_id_type=_DevIdType.MESH)
        rv = pltpu.make_async_remote_copy(
            src_ref=v_ref, dst_ref=vr_ref,
            send_sem=sems.at[2], recv_sem=sems.at[3],
            device_id=partner, device_id_type=_DevIdType.MESH)
        rk.start()
        rv.start()
        rk.wait()
        rv.wait()

        for head in range(h):
            q = q_ref[head]
            s1 = lax.dot_general(q, k_ref[head], (((1,), (1,)), ((), ())),
                                 preferred_element_type=jnp.float32) * scale
            s2 = lax.dot_general(q, kr_ref[head], (((1,), (1,)), ((), ())),
                                 preferred_element_type=jnp.float32) * scale
            m = jnp.maximum(jnp.max(s1, axis=1, keepdims=True),
                            jnp.max(s2, axis=1, keepdims=True))
            p1 = jnp.exp(s1 - m)
            p2 = jnp.exp(s2 - m)
            denom = (jnp.sum(p1, axis=1, keepdims=True)
                     + jnp.sum(p2, axis=1, keepdims=True))
            o = (lax.dot_general(p1, v_ref[head], (((1,), (0,)), ((), ())),
                                 preferred_element_type=jnp.float32)
                 + lax.dot_general(p2, vr_ref[head], (((1,), (0,)), ((), ())),
                                   preferred_element_type=jnp.float32)) / denom
            o_ref[head] = o

    out = pl.pallas_call(
        body,
        out_shape=jax.ShapeDtypeStruct((h, s, d), jnp.float32),
        in_specs=[pl.BlockSpec(memory_space=pltpu.VMEM)] * 3,
        out_specs=pl.BlockSpec(memory_space=pltpu.VMEM),
        scratch_shapes=[
            pltpu.VMEM((h, s, d), jnp.float32),
            pltpu.VMEM((h, s, d), jnp.float32),
            pltpu.SemaphoreType.DMA((4,)),
        ],
        compiler_params=pltpu.CompilerParams(collective_id=0),
    )(qt, kt, vt)
    return jnp.transpose(out, (1, 0, 2))[None]


# baseline (device time: 334055 ns/iter reference)
import jax
import jax.numpy as jnp
from jax import lax
from jax.experimental import pallas as pl
from jax.experimental.pallas import tpu as pltpu

_sem_signal = getattr(pltpu, "semaphore_signal", None) or pl.semaphore_signal
_sem_wait = getattr(pltpu, "semaphore_wait", None) or pl.semaphore_wait
_DevIdType = getattr(pltpu, "DeviceIdType", None) or pl.DeviceIdType


def kernel(Q, K, V):
    b, s, h, d = Q.shape
    scale = d ** -0.5

    qt = jnp.transpose(Q[0], (1, 0, 2))
    kt = jnp.transpose(K[0], (1, 0, 2))
    vt = jnp.transpose(V[0], (1, 0, 2))

    def body(q_ref, k_ref, v_ref, o_ref, kr_ref, vr_ref, sems):
        my_x = lax.axis_index("x")
        my_y = lax.axis_index("y")
        my_z = lax.axis_index("z")
        partner = (1 - my_x, my_y, my_z)

        barrier = pltpu.get_barrier_semaphore()
        _sem_signal(barrier, inc=1, device_id=partner,
                    device_id_type=_DevIdType.MESH)
        _sem_wait(barrier, 1)

        rk = pltpu.make_async_remote_copy(
            src_ref=k_ref, dst_ref=kr_ref,
            send_sem=sems.at[0], recv_sem=sems.at[1],
            device_id=partner, device_id_type=_DevIdType.MESH)
        rv = pltpu.make_async_remote_copy(
            src_ref=v_ref, dst_ref=vr_ref,
            send_sem=sems.at[2], recv_sem=sems.at[3],
            device_id=partner, device_id_type=_DevIdType.MESH)
        rk.start()
        rv.start()
        rk.wait()
        rv.wait()

        qb = 256

        def head_body(head, _):
            def blk_body(i, _):
                q = q_ref[head, pl.ds(i * qb, qb), :]
                s1 = lax.dot_general(q, k_ref[head], (((1,), (1,)), ((), ())),
                                     preferred_element_type=jnp.float32) * scale
                s2 = lax.dot_general(q, kr_ref[head], (((1,), (1,)), ((), ())),
                                     preferred_element_type=jnp.float32) * scale
                m = jnp.maximum(jnp.max(s1, axis=1, keepdims=True),
                                jnp.max(s2, axis=1, keepdims=True))
                p1 = jnp.exp(s1 - m)
                p2 = jnp.exp(s2 - m)
                denom = (jnp.sum(p1, axis=1, keepdims=True)
                         + jnp.sum(p2, axis=1, keepdims=True))
                o = (lax.dot_general(p1, v_ref[head], (((1,), (0,)), ((), ())),
                                     preferred_element_type=jnp.float32)
                     + lax.dot_general(p2, vr_ref[head],
                                       (((1,), (0,)), ((), ())),
                                       preferred_element_type=jnp.float32)
                     ) / denom
                o_ref[head, pl.ds(i * qb, qb), :] = o
                return 0

            return lax.fori_loop(0, s // qb, blk_body, 0)

        lax.fori_loop(0, h, head_body, 0)

    out = pl.pallas_call(
        body,
        out_shape=jax.ShapeDtypeStruct((h, s, d), jnp.float32),
        in_specs=[pl.BlockSpec(memory_space=pltpu.VMEM)] * 3,
        out_specs=pl.BlockSpec(memory_space=pltpu.VMEM),
        scratch_shapes=[
            pltpu.VMEM((h, s, d), jnp.float32),
            pltpu.VMEM((h, s, d), jnp.float32),
            pltpu.SemaphoreType.DMA((4,)),
        ],
        compiler_params=pltpu.CompilerParams(
            collective_id=0, vmem_limit_bytes=60 * 1024 * 1024),
    )(qt, kt, vt)
    return jnp.transpose(out, (1, 0, 2))[None]


# device time: 236167 ns/iter; 1.4145x vs baseline; 1.4145x over previous
import jax
import jax.numpy as jnp
from jax import lax
from jax.experimental import pallas as pl
from jax.experimental.pallas import tpu as pltpu

_sem_signal = getattr(pltpu, "semaphore_signal", None) or pl.semaphore_signal
_sem_wait = getattr(pltpu, "semaphore_wait", None) or pl.semaphore_wait
_DevIdType = getattr(pltpu, "DeviceIdType", None) or pl.DeviceIdType


def kernel(Q, K, V):
    b, s, h, d = Q.shape
    scale = d ** -0.5

    qt = jnp.transpose(Q[0], (1, 0, 2))
    kt = jnp.transpose(K[0], (1, 0, 2))
    vt = jnp.transpose(V[0], (1, 0, 2))

    def body(q_ref, k_ref, v_ref, o_ref, kr_ref, vr_ref,
             k_send, k_recv, v_send, v_recv):
        my_x = lax.axis_index("x")
        my_y = lax.axis_index("y")
        my_z = lax.axis_index("z")
        partner = (1 - my_x, my_y, my_z)

        barrier = pltpu.get_barrier_semaphore()
        _sem_signal(barrier, inc=1, device_id=partner,
                    device_id_type=_DevIdType.MESH)
        _sem_wait(barrier, 1)

        copies = []
        for head in range(h):
            rk = pltpu.make_async_remote_copy(
                src_ref=k_ref.at[head], dst_ref=kr_ref.at[head],
                send_sem=k_send.at[head], recv_sem=k_recv.at[head],
                device_id=partner, device_id_type=_DevIdType.MESH)
            rv = pltpu.make_async_remote_copy(
                src_ref=v_ref.at[head], dst_ref=vr_ref.at[head],
                send_sem=v_send.at[head], recv_sem=v_recv.at[head],
                device_id=partner, device_id_type=_DevIdType.MESH)
            rk.start()
            rv.start()
            copies.append((rk, rv))

        qb = 256

        for head in range(h):
            rk, rv = copies[head]
            rk.wait_recv()
            rv.wait_recv()

            def blk_body(i, _, head=head):
                q = q_ref[head, pl.ds(i * qb, qb), :]
                s1 = lax.dot_general(q, k_ref[head], (((1,), (1,)), ((), ())),
                                     preferred_element_type=jnp.float32) * scale
                s2 = lax.dot_general(q, kr_ref[head], (((1,), (1,)), ((), ())),
                                     preferred_element_type=jnp.float32) * scale
                m = jnp.maximum(jnp.max(s1, axis=1, keepdims=True),
                                jnp.max(s2, axis=1, keepdims=True))
                p1 = jnp.exp(s1 - m)
                p2 = jnp.exp(s2 - m)
                denom = (jnp.sum(p1, axis=1, keepdims=True)
                         + jnp.sum(p2, axis=1, keepdims=True))
                o = (lax.dot_general(p1, v_ref[head], (((1,), (0,)), ((), ())),
                                     preferred_element_type=jnp.float32)
                     + lax.dot_general(p2, vr_ref[head],
                                       (((1,), (0,)), ((), ())),
                                       preferred_element_type=jnp.float32)
                     ) / denom
                o_ref[head, pl.ds(i * qb, qb), :] = o
                return 0

            lax.fori_loop(0, s // qb, blk_body, 0)

        for rk, rv in copies:
            rk.wait_send()
            rv.wait_send()

    out = pl.pallas_call(
        body,
        out_shape=jax.ShapeDtypeStruct((h, s, d), jnp.float32),
        in_specs=[pl.BlockSpec(memory_space=pltpu.VMEM)] * 3,
        out_specs=pl.BlockSpec(memory_space=pltpu.VMEM),
        scratch_shapes=[
            pltpu.VMEM((h, s, d), jnp.float32),
            pltpu.VMEM((h, s, d), jnp.float32),
            pltpu.SemaphoreType.DMA((h,)),
            pltpu.SemaphoreType.DMA((h,)),
            pltpu.SemaphoreType.DMA((h,)),
            pltpu.SemaphoreType.DMA((h,)),
        ],
        compiler_params=pltpu.CompilerParams(
            collective_id=0, vmem_limit_bytes=60 * 1024 * 1024),
    )(qt, kt, vt)
    return jnp.transpose(out, (1, 0, 2))[None]


# device time: 178994 ns/iter; 1.8663x vs baseline; 1.3194x over previous
import jax
import jax.numpy as jnp
from jax import lax
from jax.experimental import pallas as pl
from jax.experimental.pallas import tpu as pltpu

_sem_signal = getattr(pltpu, "semaphore_signal", None) or pl.semaphore_signal
_sem_wait = getattr(pltpu, "semaphore_wait", None) or pl.semaphore_wait
_DevIdType = getattr(pltpu, "DeviceIdType", None) or pl.DeviceIdType

HG = 4


def kernel(Q, K, V):
    b, s, h, d = Q.shape
    scale = d ** -0.5

    g_out = 2 * lax.axis_index("y") + lax.axis_index("z")
    qg = jnp.transpose(
        lax.dynamic_slice_in_dim(Q[0], g_out * HG, HG, axis=1), (1, 0, 2))
    kg = jnp.transpose(
        lax.dynamic_slice_in_dim(K[0], g_out * HG, HG, axis=1), (1, 0, 2))
    vg = jnp.transpose(
        lax.dynamic_slice_in_dim(V[0], g_out * HG, HG, axis=1), (1, 0, 2))

    def body(qg_ref, kg_ref, vg_ref, o_ref, kr_ref, vr_ref,
             k_send, k_recv, v_send, v_recv,
             y_send, y_recv, z_send, z_recv):
        my_x = lax.axis_index("x")
        my_y = lax.axis_index("y")
        my_z = lax.axis_index("z")
        partner = (1 - my_x, my_y, my_z)
        y_nbr = (my_x, 1 - my_y, my_z)
        z_nbr = (my_x, my_y, 1 - my_z)

        g = 2 * my_y + my_z
        gy = 2 * (1 - my_y) + my_z
        gz = 2 * my_y + (1 - my_z)
        gd = 2 * (1 - my_y) + (1 - my_z)

        barrier = pltpu.get_barrier_semaphore()
        for nbr in (partner, y_nbr, z_nbr):
            _sem_signal(barrier, inc=1, device_id=nbr,
                        device_id_type=_DevIdType.MESH)
        _sem_wait(barrier, 3)

        rk = pltpu.make_async_remote_copy(
            src_ref=kg_ref, dst_ref=kr_ref,
            send_sem=k_send, recv_sem=k_recv,
            device_id=partner, device_id_type=_DevIdType.MESH)
        rv = pltpu.make_async_remote_copy(
            src_ref=vg_ref, dst_ref=vr_ref,
            send_sem=v_send, recv_sem=v_recv,
            device_id=partner, device_id_type=_DevIdType.MESH)
        rk.start()
        rv.start()
        rk.wait_recv()
        rv.wait_recv()

        qb = 256

        for i in range(HG):
            def blk_body(j, _, i=i):
                q = qg_ref[i, pl.ds(j * qb, qb), :]
                s1 = lax.dot_general(q, kg_ref[i], (((1,), (1,)), ((), ())),
                                     preferred_element_type=jnp.float32) * scale
                s2 = lax.dot_general(q, kr_ref[i], (((1,), (1,)), ((), ())),
                                     preferred_element_type=jnp.float32) * scale
                m = jnp.maximum(jnp.max(s1, axis=1, keepdims=True),
                                jnp.max(s2, axis=1, keepdims=True))
                p1 = jnp.exp(s1 - m)
                p2 = jnp.exp(s2 - m)
                denom = (jnp.sum(p1, axis=1, keepdims=True)
                         + jnp.sum(p2, axis=1, keepdims=True))
                o = (lax.dot_general(p1, vg_ref[i], (((1,), (0,)), ((), ())),
                                     preferred_element_type=jnp.float32)
                     + lax.dot_general(p2, vr_ref[i],
                                       (((1,), (0,)), ((), ())),
                                       preferred_element_type=jnp.float32)
                     ) / denom
                o_ref[pl.ds(g * HG + i, 1), pl.ds(j * qb, qb), :] = (
                    o[jnp.newaxis])
                return 0

            lax.fori_loop(0, s // qb, blk_body, 0)

        ry_out = pltpu.make_async_remote_copy(
            src_ref=o_ref.at[pl.ds(g * HG, HG)],
            dst_ref=o_ref.at[pl.ds(g * HG, HG)],
            send_sem=y_send, recv_sem=y_recv,
            device_id=y_nbr, device_id_type=_DevIdType.MESH)
        ry_out.start()
        ry_in = pltpu.make_async_remote_copy(
            src_ref=o_ref.at[pl.ds(gy * HG, HG)],
            dst_ref=o_ref.at[pl.ds(gy * HG, HG)],
            send_sem=y_send, recv_sem=y_recv,
            device_id=y_nbr, device_id_type=_DevIdType.MESH)
        ry_in.wait_recv()

        rz_out = []
        for slot, grp in ((0, g), (1, gy)):
            r = pltpu.make_async_remote_copy(
                src_ref=o_ref.at[pl.ds(grp * HG, HG)],
                dst_ref=o_ref.at[pl.ds(grp * HG, HG)],
                send_sem=z_send.at[slot], recv_sem=z_recv.at[slot],
                device_id=z_nbr, device_id_type=_DevIdType.MESH)
            r.start()
            rz_out.append(r)
        for slot, grp in ((0, gz), (1, gd)):
            r = pltpu.make_async_remote_copy(
                src_ref=o_ref.at[pl.ds(grp * HG, HG)],
                dst_ref=o_ref.at[pl.ds(grp * HG, HG)],
                send_sem=z_send.at[slot], recv_sem=z_recv.at[slot],
                device_id=z_nbr, device_id_type=_DevIdType.MESH)
            r.wait_recv()

        rk.wait_send()
        rv.wait_send()
        ry_out.wait_send()
        for r in rz_out:
            r.wait_send()

    out = pl.pallas_call(
        body,
        out_shape=jax.ShapeDtypeStruct((h, s, d), jnp.float32),
        in_specs=[pl.BlockSpec(memory_space=pltpu.VMEM)] * 3,
        out_specs=pl.BlockSpec(memory_space=pltpu.VMEM),
        scratch_shapes=[
            pltpu.VMEM((HG, s, d), jnp.float32),
            pltpu.VMEM((HG, s, d), jnp.float32),
            pltpu.SemaphoreType.DMA,
            pltpu.SemaphoreType.DMA,
            pltpu.SemaphoreType.DMA,
            pltpu.SemaphoreType.DMA,
            pltpu.SemaphoreType.DMA,
            pltpu.SemaphoreType.DMA,
            pltpu.SemaphoreType.DMA((2,)),
            pltpu.SemaphoreType.DMA((2,)),
        ],
        compiler_params=pltpu.CompilerParams(
            collective_id=0, vmem_limit_bytes=60 * 1024 * 1024),
    )(qg, kg, vg)
    return jnp.transpose(out, (1, 0, 2))[None]


# device time: 117364 ns/iter; 2.8463x vs baseline; 1.5251x over previous
import jax
import jax.numpy as jnp
from jax import lax
from jax.experimental import pallas as pl
from jax.experimental.pallas import tpu as pltpu

_sem_signal = getattr(pltpu, "semaphore_signal", None) or pl.semaphore_signal
_sem_wait = getattr(pltpu, "semaphore_wait", None) or pl.semaphore_wait
_DevIdType = getattr(pltpu, "DeviceIdType", None) or pl.DeviceIdType

HG = 4


def kernel(Q, K, V):
    b, s, h, d = Q.shape
    scale = d ** -0.5

    g_out = 2 * lax.axis_index("y") + lax.axis_index("z")
    qg = jnp.transpose(
        lax.dynamic_slice_in_dim(Q[0], g_out * HG, HG, axis=1), (1, 0, 2))
    kg = jnp.transpose(
        lax.dynamic_slice_in_dim(K[0], g_out * HG, HG, axis=1), (1, 0, 2))
    vg = jnp.transpose(
        lax.dynamic_slice_in_dim(V[0], g_out * HG, HG, axis=1), (1, 0, 2))

    def body(qg_ref, kg_ref, vg_ref, o_ref, kr_ref, vr_ref,
             k_send, k_recv, v_send, v_recv,
             y_send, y_recv, z_send, z_recv):
        my_x = lax.axis_index("x")
        my_y = lax.axis_index("y")
        my_z = lax.axis_index("z")
        partner = (1 - my_x, my_y, my_z)
        y_nbr = (my_x, 1 - my_y, my_z)
        z_nbr = (my_x, my_y, 1 - my_z)

        g = 2 * my_y + my_z
        gy = 2 * (1 - my_y) + my_z
        gz = 2 * my_y + (1 - my_z)
        gd = 2 * (1 - my_y) + (1 - my_z)

        def head_copy(src, dst, idx_s, idx_d, ssem, rsem, dev):
            return pltpu.make_async_remote_copy(
                src_ref=src.at[idx_s] if idx_s is not None else src,
                dst_ref=dst.at[idx_d] if idx_d is not None else dst,
                send_sem=ssem, recv_sem=rsem,
                device_id=dev, device_id_type=_DevIdType.MESH)

        barrier = pltpu.get_barrier_semaphore()
        for nbr in (partner, y_nbr, z_nbr):
            _sem_signal(barrier, inc=1, device_id=nbr,
                        device_id_type=_DevIdType.MESH)
        _sem_wait(barrier, 3)

        kv_copies = []
        for i in range(HG):
            rk = head_copy(kg_ref, kr_ref, i, i,
                           k_send.at[i], k_recv.at[i], partner)
            rv = head_copy(vg_ref, vr_ref, i, i,
                           v_send.at[i], v_recv.at[i], partner)
            rk.start()
            rv.start()
            kv_copies.append((rk, rv))

        qb = 512

        def out_slice(grp, i):
            return pl.ds((grp * HG + i) * 1, 1)

        ry_out, rz_out = [], []
        for i in range(HG):
            rk, rv = kv_copies[i]
            rk.wait_recv()
            rv.wait_recv()

            def blk_body(j, _, i=i):
                q = qg_ref[i, pl.ds(j * qb, qb), :]
                s1 = lax.dot_general(q, kg_ref[i], (((1,), (1,)), ((), ())),
                                     preferred_element_type=jnp.float32) * scale
                s2 = lax.dot_general(q, kr_ref[i], (((1,), (1,)), ((), ())),
                                     preferred_element_type=jnp.float32) * scale
                m = jnp.maximum(jnp.max(s1, axis=1, keepdims=True),
                                jnp.max(s2, axis=1, keepdims=True))
                p1 = jnp.exp(s1 - m)
                p2 = jnp.exp(s2 - m)
                denom = (jnp.sum(p1, axis=1, keepdims=True)
                         + jnp.sum(p2, axis=1, keepdims=True))
                o = (lax.dot_general(p1, vg_ref[i], (((1,), (0,)), ((), ())),
                                     preferred_element_type=jnp.float32)
                     + lax.dot_general(p2, vr_ref[i],
                                       (((1,), (0,)), ((), ())),
                                       preferred_element_type=jnp.float32)
                     ) / denom
                o_ref[out_slice(g, i), pl.ds(j * qb, qb), :] = o[jnp.newaxis]
                return 0

            lax.fori_loop(0, s // qb, blk_body, 0)

            ry = head_copy(o_ref, o_ref, out_slice(g, i), out_slice(g, i),
                           y_send.at[i], y_recv.at[i], y_nbr)
            ry.start()
            ry_out.append(ry)
            rz = head_copy(o_ref, o_ref, out_slice(g, i), out_slice(g, i),
                           z_send.at[0, i], z_recv.at[0, i], z_nbr)
            rz.start()
            rz_out.append(rz)

        for i in range(HG):
            ry_in = head_copy(o_ref, o_ref, out_slice(gy, i), out_slice(gy, i),
                              y_send.at[i], y_recv.at[i], y_nbr)
            ry_in.wait_recv()
            rz = head_copy(o_ref, o_ref, out_slice(gy, i), out_slice(gy, i),
                           z_send.at[1, i], z_recv.at[1, i], z_nbr)
            rz.start()
            rz_out.append(rz)

        for slot, grp in ((0, gz), (1, gd)):
            for i in range(HG):
                rz_in = head_copy(o_ref, o_ref, out_slice(grp, i),
                                  out_slice(grp, i),
                                  z_send.at[slot, i], z_recv.at[slot, i],
                                  z_nbr)
                rz_in.wait_recv()

        for rk, rv in kv_copies:
            rk.wait_send()
            rv.wait_send()
        for r in ry_out:
            r.wait_send()
        for r in rz_out:
            r.wait_send()

    out = pl.pallas_call(
        body,
        out_shape=jax.ShapeDtypeStruct((h, s, d), jnp.float32),
        in_specs=[pl.BlockSpec(memory_space=pltpu.VMEM)] * 3,
        out_specs=pl.BlockSpec(memory_space=pltpu.VMEM),
        scratch_shapes=[
            pltpu.VMEM((HG, s, d), jnp.float32),
            pltpu.VMEM((HG, s, d), jnp.float32),
            pltpu.SemaphoreType.DMA((HG,)),
            pltpu.SemaphoreType.DMA((HG,)),
            pltpu.SemaphoreType.DMA((HG,)),
            pltpu.SemaphoreType.DMA((HG,)),
            pltpu.SemaphoreType.DMA((HG,)),
            pltpu.SemaphoreType.DMA((HG,)),
            pltpu.SemaphoreType.DMA((2, HG)),
            pltpu.SemaphoreType.DMA((2, HG)),
        ],
        compiler_params=pltpu.CompilerParams(
            collective_id=0, vmem_limit_bytes=60 * 1024 * 1024),
    )(qg, kg, vg)
    return jnp.transpose(out, (1, 0, 2))[None]


# device time: 108141 ns/iter; 3.0891x vs baseline; 1.0853x over previous
import jax
import jax.numpy as jnp
from jax import lax
from jax.experimental import pallas as pl
from jax.experimental.pallas import tpu as pltpu

_sem_signal = getattr(pltpu, "semaphore_signal", None) or pl.semaphore_signal
_sem_wait = getattr(pltpu, "semaphore_wait", None) or pl.semaphore_wait
_DevIdType = getattr(pltpu, "DeviceIdType", None) or pl.DeviceIdType

HG = 4


def kernel(Q, K, V):
    b, s, h, d = Q.shape
    scale = d ** -0.5

    def body(q_ref, k_ref, v_ref, o_ref,
             qs_ref, ks_ref, vs_ref, os_ref, kr_ref, vr_ref,
             gat_sem, st_sem,
             k_send, k_recv, v_send, v_recv,
             y_send, y_recv, z_send, z_recv):
        my_x = lax.axis_index("x")
        my_y = lax.axis_index("y")
        my_z = lax.axis_index("z")
        partner = (1 - my_x, my_y, my_z)
        y_nbr = (my_x, 1 - my_y, my_z)
        z_nbr = (my_x, my_y, 1 - my_z)

        g = 2 * my_y + my_z
        gy = 2 * (1 - my_y) + my_z
        gz = 2 * my_y + (1 - my_z)
        gd = 2 * (1 - my_y) + (1 - my_z)

        gathers = []
        for i in range(HG):
            hd = g * HG + i
            trips = []
            for slot, (src, dst) in enumerate(
                    ((q_ref, qs_ref), (k_ref, ks_ref), (v_ref, vs_ref))):
                c = pltpu.make_async_copy(
                    src.at[:, hd, :], dst.at[i], gat_sem.at[slot, i])
                c.start()
                trips.append(c)
            gathers.append(trips)

        barrier = pltpu.get_barrier_semaphore()
        for nbr in (partner, y_nbr, z_nbr):
            _sem_signal(barrier, inc=1, device_id=nbr,
                        device_id_type=_DevIdType.MESH)
        _sem_wait(barrier, 3)

        def rdma(src, dst, ssem, rsem, dev):
            return pltpu.make_async_remote_copy(
                src_ref=src, dst_ref=dst, send_sem=ssem, recv_sem=rsem,
                device_id=dev, device_id_type=_DevIdType.MESH)

        kv_copies = []
        for i in range(HG):
            hd = g * HG + i
            rk = rdma(k_ref.at[:, hd, :], kr_ref.at[i],
                      k_send.at[i], k_recv.at[i], partner)
            rv = rdma(v_ref.at[:, hd, :], vr_ref.at[i],
                      v_send.at[i], v_recv.at[i], partner)
            rk.start()
            rv.start()
            kv_copies.append((rk, rv))

        qb = 512

        ry_out, rz_out, st_copies = [], [], []
        for i in range(HG):
            for c in gathers[i]:
                c.wait()
            rk, rv = kv_copies[i]
            rk.wait_recv()
            rv.wait_recv()

            def blk_body(j, _, i=i):
                q = qs_ref[i, pl.ds(j * qb, qb), :]
                s1 = lax.dot_general(q, ks_ref[i], (((1,), (1,)), ((), ())),
                                     preferred_element_type=jnp.float32) * scale
                s2 = lax.dot_general(q, kr_ref[i], (((1,), (1,)), ((), ())),
                                     preferred_element_type=jnp.float32) * scale
                m = jnp.maximum(jnp.max(s1, axis=1, keepdims=True),
                                jnp.max(s2, axis=1, keepdims=True))
                p1 = jnp.exp(s1 - m)
                p2 = jnp.exp(s2 - m)
                denom = (jnp.sum(p1, axis=1, keepdims=True)
                         + jnp.sum(p2, axis=1, keepdims=True))
                o = (lax.dot_general(p1, vs_ref[i], (((1,), (0,)), ((), ())),
                                     preferred_element_type=jnp.float32)
                     + lax.dot_general(p2, vr_ref[i],
                                       (((1,), (0,)), ((), ())),
                                       preferred_element_type=jnp.float32)
                     ) / denom
                os_ref[i, pl.ds(j * qb, qb), :] = o
                return 0

            lax.fori_loop(0, s // qb, blk_body, 0)

            hd = g * HG + i
            st = pltpu.make_async_copy(
                os_ref.at[i], o_ref.at[:, hd, :], st_sem.at[i])
            st.start()
            st_copies.append(st)
            ry = rdma(os_ref.at[i], o_ref.at[:, hd, :],
                      y_send.at[i], y_recv.at[i], y_nbr)
            ry.start()
            ry_out.append(ry)
            rz = rdma(os_ref.at[i], o_ref.at[:, hd, :],
                      z_send.at[0, i], z_recv.at[0, i], z_nbr)
            rz.start()
            rz_out.append(rz)

        for i in range(HG):
            hd = gy * HG + i
            ry_in = rdma(o_ref.at[:, hd, :], o_ref.at[:, hd, :],
                         y_send.at[i], y_recv.at[i], y_nbr)
            ry_in.wait_recv()
            rz = rdma(o_ref.at[:, hd, :], o_ref.at[:, hd, :],
                      z_send.at[1, i], z_recv.at[1, i], z_nbr)
            rz.start()
            rz_out.append(rz)

        for slot, grp in ((0, gz), (1, gd)):
            for i in range(HG):
                hd = grp * HG + i
                rz_in = rdma(o_ref.at[:, hd, :], o_ref.at[:, hd, :],
                             z_send.at[slot, i], z_recv.at[slot, i], z_nbr)
                rz_in.wait_recv()

        for rk, rv in kv_copies:
            rk.wait_send()
            rv.wait_send()
        for r in ry_out:
            r.wait_send()
        for r in rz_out:
            r.wait_send()
        for c in st_copies:
            c.wait()

    out = pl.pallas_call(
        body,
        out_shape=jax.ShapeDtypeStruct((s, h, d), jnp.float32),
        in_specs=[pl.BlockSpec(memory_space=pltpu.VMEM)] * 3,
        out_specs=pl.BlockSpec(memory_space=pltpu.VMEM),
        scratch_shapes=[
            pltpu.VMEM((HG, s, d), jnp.float32),
            pltpu.VMEM((HG, s, d), jnp.float32),
            pltpu.VMEM((HG, s, d), jnp.float32),
            pltpu.VMEM((HG, s, d), jnp.float32),
            pltpu.VMEM((HG, s, d), jnp.float32),
            pltpu.VMEM((HG, s, d), jnp.float32),
            pltpu.SemaphoreType.DMA((3, HG)),
            pltpu.SemaphoreType.DMA((HG,)),
            pltpu.SemaphoreType.DMA((HG,)),
            pltpu.SemaphoreType.DMA((HG,)),
            pltpu.SemaphoreType.DMA((HG,)),
            pltpu.SemaphoreType.DMA((HG,)),
            pltpu.SemaphoreType.DMA((HG,)),
            pltpu.SemaphoreType.DMA((HG,)),
            pltpu.SemaphoreType.DMA((2, HG)),
            pltpu.SemaphoreType.DMA((2, HG)),
        ],
        compiler_params=pltpu.CompilerParams(
            collective_id=0, vmem_limit_bytes=62 * 1024 * 1024),
    )(Q[0], K[0], V[0])
    return out[jnp.newaxis]
